# baseline (speedup 1.0000x reference)
import functools

import jax
import jax.numpy as jnp
from jax import lax
from jax.experimental import pallas as pl
from jax.experimental.pallas import tpu as pltpu
from jax.experimental.pallas import tpu_sc as plsc

_NC = 2
_NS = 16
_NW = _NC * _NS
_L = 16

_CHUNK = 2048
_PCHUNK = 4096


def _rsqrt(x):
    i = lax.bitcast_convert_type(x, jnp.int32)
    i = jnp.int32(0x5F3759DF) - lax.shift_right_arithmetic(i, 1)
    y = lax.bitcast_convert_type(i, jnp.float32)
    for _ in range(2):
        y = y * (1.5 - 0.5 * x * y * y)
    return y


def _build_body(hw, n_chunks, j_hbm, tab_hbm, *scratch):
    (p0_a, p1_a, p2_a, t_a, p0_b, p1_b, p2_b, t_b,
     sem_in_a, sem_in_b, sem_o_a, sem_o_b) = scratch
    bufs = (
        dict(pv=(p0_a, p1_a, p2_a), t=t_a, sem_in=sem_in_a, sem_o=sem_o_a),
        dict(pv=(p0_b, p1_b, p2_b), t=t_b, sem_in=sem_in_b, sem_o=sem_o_b),
    )
    wid = lax.axis_index("s") * _NC + lax.axis_index("c")
    base0 = wid * (n_chunks * _PCHUNK)

    iota = lax.iota(jnp.int32, _L)
    pat = lax.shift_left(lax.shift_right_logical(iota, 1), 3) \
        + jnp.bitwise_and(iota, 1) * 3
    pats = [pat + c for c in range(3)]

    def fire_in(k, b):
        q0 = base0 + k * _PCHUNK
        for c in range(3):
            pltpu.async_copy(j_hbm.at[pl.ds(c * hw + q0, _PCHUNK)],
                             b["pv"][c], b["sem_in"])

    def wait_in(b):
        for c in range(3):
            pltpu.make_async_copy(j_hbm.at[pl.ds(0, _PCHUNK)],
                                  b["pv"][c], b["sem_in"]).wait()

    def interleave(b):
        pv, tbuf = b["pv"], b["t"]

        def ib(i, _):
            s = i * _L
            off = s * 4
            for c in range(3):
                plsc.store_scatter(tbuf, [pats[c] + off],
                                   pv[c][pl.ds(s, _L)])
            return ()

        lax.fori_loop(0, _PCHUNK // _L, ib, (), unroll=4)

    def fire_out(k, b):
        q0 = base0 + k * _PCHUNK
        pltpu.async_copy(b["t"], tab_hbm.at[pl.ds(4 * q0, 4 * _PCHUNK)],
                         b["sem_o"])

    def wait_out(b):
        pltpu.make_async_copy(b["t"], tab_hbm.at[pl.ds(0, 4 * _PCHUNK)],
                              b["sem_o"]).wait()

    fire_in(0, bufs[0])

    def step(k, p, not_first2, not_last):
        b, o = bufs[p], bufs[1 - p]
        wait_in(b)
        if not_last is True:
            fire_in(k + 1, o)
        elif not_last is not False:
            @pl.when(not_last)
            def _():
                fire_in(k + 1, o)
        if not_first2 is True:
            wait_out(b)
        elif not_first2 is not False:
            @pl.when(not_first2)
            def _():
                wait_out(b)
        interleave(b)
        fire_out(k, b)

    def pair_body(j, _):
        step(j * 2, 0, j >= 1, True)
        step(j * 2 + 1, 1, j >= 1, j < n_chunks // 2 - 1)
        return ()

    assert n_chunks % 2 == 0
    lax.fori_loop(0, n_chunks // 2, pair_body, ())
    wait_out(bufs[0])
    wait_out(bufs[1])


def _gather_body(n, n_per_w, n_chunks, width, height,
                 u_hbm, v_hbm, cp0_hbm, cp1_hbm, cp2_hbm, tab_hbm,
                 consts_hbm, out_hbm, *scratch):
    (u_a, v_a, cp0_a, cp1_a, cp2_a, idx_a, sub_a, rows_a, out_a,
     u_b, v_b, cp0_b, cp1_b, cp2_b, idx_b, sub_b, rows_b, out_b,
     consts_v, sem_in_a, sem_in_b, sem_g_a, sem_g_b,
     sem_o_a, sem_o_b) = scratch
    bufs = (
        dict(u=u_a, v=v_a, cp=(cp0_a, cp1_a, cp2_a), idx=idx_a, sub=sub_a,
             rows=rows_a, out=out_a, sem_in=sem_in_a, sem_g=sem_g_a,
             sem_o=sem_o_a),
        dict(u=u_b, v=v_b, cp=(cp0_b, cp1_b, cp2_b), idx=idx_b, sub=sub_b,
             rows=rows_b, out=out_b, sem_in=sem_in_b, sem_g=sem_g_b,
             sem_o=sem_o_b),
    )
    cp_hbms = (cp0_hbm, cp1_hbm, cp2_hbm)

    wid = lax.axis_index("s") * _NC + lax.axis_index("c")
    base0 = wid * n_per_w

    pltpu.sync_copy(consts_hbm, consts_v)
    nbeta = [consts_v[pl.ds(c * _L, _L)] for c in range(3)]
    ngamma = [consts_v[pl.ds((3 + c) * _L, _L)] for c in range(3)]
    bc = [consts_v[pl.ds((6 + c) * _L, _L)] for c in range(3)]

    lg_w8 = (8 * width).bit_length() - 1
    iota = lax.iota(jnp.int32, _L)

    def fire_in(k, b):
        base = base0 + k * _CHUNK
        pltpu.async_copy(u_hbm.at[pl.ds(base, _CHUNK)], b["u"], b["sem_in"])
        pltpu.async_copy(v_hbm.at[pl.ds(base, _CHUNK)], b["v"], b["sem_in"])
        for c in range(3):
            pltpu.async_copy(cp_hbms[c].at[pl.ds(base, _CHUNK)],
                             b["cp"][c], b["sem_in"])

    def wait_in(b):
        pltpu.make_async_copy(u_hbm.at[pl.ds(0, _CHUNK)], b["u"],
                              b["sem_in"]).wait()
        pltpu.make_async_copy(v_hbm.at[pl.ds(0, _CHUNK)], b["v"],
                              b["sem_in"]).wait()
        for c in range(3):
            pltpu.make_async_copy(cp_hbms[c].at[pl.ds(0, _CHUNK)],
                                  b["cp"][c], b["sem_in"]).wait()

    def do_idx(b):
        u_v, v_v, idx_v, sub_v = b["u"], b["v"], b["idx"], b["sub"]

        def idx_body(i, _):
            s = i * _L
            uu = u_v[pl.ds(s, _L)]
            vv = v_v[pl.ds(s, _L)]
            t = (lax.shift_left(lax.shift_right_logical(vv, 3), lg_w8)
                 + lax.shift_left(lax.shift_right_logical(uu, 7), 10)
                 + lax.shift_left(jnp.bitwise_and(vv, 7), 7)
                 + jnp.bitwise_and(uu, 127))
            idx_v[pl.ds(s, _L)] = lax.shift_right_logical(t, 1)
            sub_v[pl.ds(s, _L)] = jnp.bitwise_and(t, 1) * 3
            return ()

        lax.fori_loop(0, _CHUNK // _L, idx_body, (), unroll=4)

    def fire_gather(b):
        pltpu.async_copy(tab_hbm.at[b["idx"]], b["rows"], b["sem_g"])

    def wait_gather(b):
        pltpu.make_async_copy(tab_hbm.at[b["idx"]], b["rows"],
                              b["sem_g"]).wait()

    def do_compute(b):
        cp0_v, cp1_v, cp2_v = b["cp"]
        rows_v, out_v, sub_v = b["rows"], b["out"], b["sub"]

        def comp_body(i, _):
            s = i * _L
            x0 = cp0_v[pl.ds(s, _L)]
            x1 = cp1_v[pl.ds(s, _L)]
            x2 = cp2_v[pl.ds(s, _L)]
            z2 = x0 * x0 + x1 * x1 + x2 * x2
            z = z2 * _rsqrt(jnp.maximum(z2, jnp.float32(1e-30)))
            sub16 = sub_v[pl.ds(s, _L)]
            row16 = s + iota
            po = lax.shift_left(lax.shift_right_logical(s, 7), 9) \
                + jnp.bitwise_and(s, 127)
            for c in range(3):
                g = plsc.load_gather(rows_v, [row16, sub16 + c])
                e1 = jnp.exp(nbeta[c] * z)
                e2 = jnp.exp(ngamma[c] * z)
                out_v[pl.ds(po + c * 128, _L)] = g * e1 + bc[c] * (1.0 - e2)
            return ()

        lax.fori_loop(0, _CHUNK // _L, comp_body, (), unroll=4)

    def fire_out(k, b):
        base = base0 + k * _CHUNK
        pltpu.async_copy(b["out"], out_hbm.at[pl.ds(4 * base, 4 * _CHUNK)],
                         b["sem_o"])

    def wait_out(b):
        pltpu.make_async_copy(b["out"],
                              out_hbm.at[pl.ds(0, 4 * _CHUNK)],
                              b["sem_o"]).wait()

    fire_in(0, bufs[0])

    def step(k, p, jge1, jge2, jlt_last):
        b, o = bufs[p], bufs[1 - p]
        wait_in(b)
        do_idx(b)
        fire_gather(b)
        if jge2 is True:
            wait_out(o)
        elif jge2 is not False:
            @pl.when(jge2)
            def _():
                wait_out(o)
        if jge1 is True:
            wait_gather(o)
            do_compute(o)
            fire_out(k - 1, o)
        elif jge1 is not False:
            @pl.when(jge1)
            def _():
                wait_gather(o)
                do_compute(o)
                fire_out(k - 1, o)
        if jlt_last is True:
            fire_in(k + 1, o)
        elif jlt_last is not False:
            @pl.when(jlt_last)
            def _():
                fire_in(k + 1, o)

    def pair_body(j, _):
        k0 = j * 2
        step(k0, 0, j >= 1, j >= 2, True)
        step(k0 + 1, 1, True, j >= 1, j < n_chunks // 2 - 1)
        return ()

    assert n_chunks % 2 == 0
    lax.fori_loop(0, n_chunks // 2, pair_body, ())

    last = n_chunks - 1
    b_last = bufs[last & 1]
    wait_out(b_last)
    wait_gather(b_last)
    do_compute(b_last)
    fire_out(last, b_last)
    wait_out(bufs[(last - 1) & 1])
    wait_out(b_last)


def kernel(u, v, cP, J, B, beta, gamma):
    n = u.shape[0]
    h, w, _ = J.shape
    hw = h * w
    assert n % (_NW * _CHUNK) == 0 and n % 128 == 0
    assert hw % (_NW * _PCHUNK) == 0
    n_per_w = n // _NW
    n_chunks = n_per_w // _CHUNK
    p_chunks = hw // _NW // _PCHUNK

    assert h % 8 == 0 and w % 128 == 0 and (w & (w - 1)) == 0
    j_flat = (J.transpose(2, 0, 1)
               .reshape(3, h // 8, 8, w // 128, 128)
               .transpose(0, 1, 3, 2, 4)
               .reshape(3 * hw))
    consts = jnp.broadcast_to(
        jnp.concatenate([-beta, -gamma, B], axis=0), (9, _L)
    ).astype(jnp.float32).reshape(9 * _L)

    mesh = plsc.VectorSubcoreMesh(core_axis_name="c", subcore_axis_name="s")
    cparams = pltpu.CompilerParams(
        needs_layout_passes=False, use_tc_tiling_on_sc=False)

    pbuf_set = [
        pltpu.VMEM((_PCHUNK,), jnp.float32),
        pltpu.VMEM((_PCHUNK,), jnp.float32),
        pltpu.VMEM((_PCHUNK,), jnp.float32),
        pltpu.VMEM((4 * _PCHUNK,), jnp.float32),
    ]
    build_fn = pl.kernel(
        functools.partial(_build_body, hw, p_chunks),
        out_type=jax.ShapeDtypeStruct((4 * hw,), jnp.float32),
        mesh=mesh,
        compiler_params=cparams,
        scratch_types=(pbuf_set + pbuf_set
                       + [pltpu.SemaphoreType.DMA] * 4),
    )
    tab = build_fn(j_flat)

    buf_set = [
        pltpu.VMEM((_CHUNK,), jnp.int32),
        pltpu.VMEM((_CHUNK,), jnp.int32),
        pltpu.VMEM((_CHUNK,), jnp.float32),
        pltpu.VMEM((_CHUNK,), jnp.float32),
        pltpu.VMEM((_CHUNK,), jnp.float32),
        pltpu.VMEM((_CHUNK,), jnp.int32),
        pltpu.VMEM((_CHUNK,), jnp.int32),
        pltpu.VMEM((_CHUNK, 8), jnp.float32),
        pltpu.VMEM((4 * _CHUNK,), jnp.float32),
    ]
    gather_fn = pl.kernel(
        functools.partial(_gather_body, n, n_per_w, n_chunks, w, h),
        out_type=jax.ShapeDtypeStruct((4 * n,), jnp.float32),
        mesh=mesh,
        compiler_params=cparams,
        scratch_types=(
            buf_set + buf_set
            + [pltpu.VMEM((9 * _L,), jnp.float32)]
            + [pltpu.SemaphoreType.DMA] * 6
        ),
    )
    out_phys = gather_fn(u.astype(jnp.int32), v.astype(jnp.int32),
                         cP[0], cP[1], cP[2], tab.reshape(hw // 2, 8),
                         consts)
    return (out_phys.reshape(n // 128, 4, 128)
            .transpose(1, 0, 2)
            .reshape(4, n)[:3])


# speedup vs baseline: 1.1400x; 1.1400x over previous
import functools

import jax
import jax.numpy as jnp
from jax import lax
from jax.experimental import pallas as pl
from jax.experimental.pallas import tpu as pltpu
from jax.experimental.pallas import tpu_sc as plsc

_NC = 2
_NS = 16
_NW = _NC * _NS
_L = 16

_CHUNK = 4096


def _rsqrt(x):
    i = lax.bitcast_convert_type(x, jnp.int32)
    i = jnp.int32(0x5F3759DF) - lax.shift_right_arithmetic(i, 1)
    y = lax.bitcast_convert_type(i, jnp.float32)
    for _ in range(2):
        y = y * (1.5 - 0.5 * x * y * y)
    return y


def _sc_body(n, n_per_w, n_chunks, width, height,
             u_hbm, v_hbm, cp0_hbm, cp1_hbm, cp2_hbm, j_hbm, consts_hbm,
             out_hbm, *scratch):
    (u_a, v_a, cp0_a, cp1_a, cp2_a, idx_a, rows_a, out_a,
     u_b, v_b, cp0_b, cp1_b, cp2_b, idx_b, rows_b, out_b,
     consts_v, sem_in_a, sem_in_b, sem_g_a, sem_g_b,
     sem_o_a, sem_o_b) = scratch
    bufs = (
        dict(u=u_a, v=v_a, cp=(cp0_a, cp1_a, cp2_a), idx=idx_a,
             rows=rows_a, out=out_a, sem_in=sem_in_a, sem_g=sem_g_a,
             sem_o=sem_o_a),
        dict(u=u_b, v=v_b, cp=(cp0_b, cp1_b, cp2_b), idx=idx_b,
             rows=rows_b, out=out_b, sem_in=sem_in_b, sem_g=sem_g_b,
             sem_o=sem_o_b),
    )
    cp_hbms = (cp0_hbm, cp1_hbm, cp2_hbm)

    wid = lax.axis_index("s") * _NC + lax.axis_index("c")
    base0 = wid * n_per_w

    pltpu.sync_copy(consts_hbm, consts_v)
    nbeta = [consts_v[pl.ds(c * _L, _L)] for c in range(3)]
    ngamma = [consts_v[pl.ds((3 + c) * _L, _L)] for c in range(3)]
    bc = [consts_v[pl.ds((6 + c) * _L, _L)] for c in range(3)]

    hw = width * height
    lg_w8 = (8 * width).bit_length() - 1

    def fire_in(k, b):
        base = base0 + k * _CHUNK
        pltpu.async_copy(u_hbm.at[pl.ds(base, _CHUNK)], b["u"], b["sem_in"])
        pltpu.async_copy(v_hbm.at[pl.ds(base, _CHUNK)], b["v"], b["sem_in"])
        for c in range(3):
            pltpu.async_copy(cp_hbms[c].at[pl.ds(base, _CHUNK)],
                             b["cp"][c], b["sem_in"])

    def wait_in(b):
        pltpu.make_async_copy(u_hbm.at[pl.ds(0, _CHUNK)], b["u"],
                              b["sem_in"]).wait()
        pltpu.make_async_copy(v_hbm.at[pl.ds(0, _CHUNK)], b["v"],
                              b["sem_in"]).wait()
        for c in range(3):
            pltpu.make_async_copy(cp_hbms[c].at[pl.ds(0, _CHUNK)],
                                  b["cp"][c], b["sem_in"]).wait()

    def do_idx(b):
        u_v, v_v, idx_v = b["u"], b["v"], b["idx"]

        def idx_body(i):
            s = i * _L
            uu = u_v[pl.ds(s, _L)]
            vv = v_v[pl.ds(s, _L)]
            t = (lax.shift_left(lax.shift_right_logical(vv, 3), lg_w8)
                 + lax.shift_left(lax.shift_right_logical(uu, 7), 10)
                 + lax.shift_left(jnp.bitwise_and(vv, 7), 7)
                 + jnp.bitwise_and(uu, 127))
            idx_v[pl.ds(s, _L)] = t
            idx_v[pl.ds(_CHUNK + s, _L)] = t + hw
            idx_v[pl.ds(2 * _CHUNK + s, _L)] = t + 2 * hw

        plsc.parallel_loop(0, _CHUNK // _L, 1, unroll=8)(idx_body)

    def fire_gather(b):
        pltpu.async_copy(j_hbm.at[b["idx"]], b["rows"], b["sem_g"])

    def wait_gather(b):
        pltpu.make_async_copy(j_hbm.at[b["idx"]], b["rows"],
                              b["sem_g"]).wait()

    def do_compute(b):
        cp0_v, cp1_v, cp2_v = b["cp"]
        rows_v, out_v = b["rows"], b["out"]

        def comp_body(i):
            s = i * _L
            x0 = cp0_v[pl.ds(s, _L)]
            x1 = cp1_v[pl.ds(s, _L)]
            x2 = cp2_v[pl.ds(s, _L)]
            z2 = x0 * x0 + x1 * x1 + x2 * x2
            z = z2 * _rsqrt(jnp.maximum(z2, jnp.float32(1e-30)))
            po = lax.shift_left(lax.shift_right_logical(s, 7), 9) \
                + jnp.bitwise_and(s, 127)
            for c in range(3):
                g = rows_v[pl.ds(c * _CHUNK + s, _L)]
                e1 = jnp.exp(nbeta[c] * z)
                e2 = jnp.exp(ngamma[c] * z)
                out_v[pl.ds(po + c * 128, _L)] = g * e1 + bc[c] * (1.0 - e2)

        plsc.parallel_loop(0, _CHUNK // _L, 1, unroll=8)(comp_body)

    def fire_out(k, b):
        base = base0 + k * _CHUNK
        pltpu.async_copy(b["out"], out_hbm.at[pl.ds(4 * base, 4 * _CHUNK)],
                         b["sem_o"])

    def wait_out(b):
        pltpu.make_async_copy(b["out"],
                              out_hbm.at[pl.ds(0, 4 * _CHUNK)],
                              b["sem_o"]).wait()

    fire_in(0, bufs[0])

    def step(k, p, jge1, jge2, jlt_last):
        b, o = bufs[p], bufs[1 - p]
        wait_in(b)
        do_idx(b)
        fire_gather(b)
        if jge2 is True:
            wait_out(o)
        elif jge2 is not False:
            @pl.when(jge2)
            def _():
                wait_out(o)
        if jge1 is True:
            wait_gather(o)
            do_compute(o)
            fire_out_k = k - 1
            fire_out(fire_out_k, o)
        elif jge1 is not False:
            @pl.when(jge1)
            def _():
                wait_gather(o)
                do_compute(o)
                fire_out(k - 1, o)
        if jlt_last is True:
            fire_in(k + 1, o)
        elif jlt_last is not False:
            @pl.when(jlt_last)
            def _():
                fire_in(k + 1, o)

    def pair_body(j, _):
        k0 = j * 2
        step(k0, 0, j >= 1, j >= 2, True)
        step(k0 + 1, 1, True, j >= 1, j < n_chunks // 2 - 1)
        return ()

    assert n_chunks % 2 == 0
    lax.fori_loop(0, n_chunks // 2, pair_body, ())

    last = n_chunks - 1
    b_last = bufs[last & 1]
    wait_out(b_last)
    wait_gather(b_last)
    do_compute(b_last)
    fire_out(last, b_last)
    wait_out(bufs[(last - 1) & 1])
    wait_out(b_last)


def kernel(u, v, cP, J, B, beta, gamma):
    n = u.shape[0]
    h, w, _ = J.shape
    assert n % (_NW * _CHUNK) == 0 and n % 128 == 0
    n_per_w = n // _NW
    n_chunks = n_per_w // _CHUNK

    assert h % 8 == 0 and w % 128 == 0 and (w & (w - 1)) == 0
    j_flat = (J.transpose(2, 0, 1)
               .reshape(3, h // 8, 8, w // 128, 128)
               .transpose(0, 1, 3, 2, 4)
               .reshape(3 * h * w))
    consts = jnp.broadcast_to(
        jnp.concatenate([-beta, -gamma, B], axis=0), (9, _L)
    ).astype(jnp.float32).reshape(9 * _L)

    mesh = plsc.VectorSubcoreMesh(core_axis_name="c", subcore_axis_name="s")
    body = functools.partial(_sc_body, n, n_per_w, n_chunks, w, h)
    buf_set = [
        pltpu.VMEM((_CHUNK,), jnp.int32),
        pltpu.VMEM((_CHUNK,), jnp.int32),
        pltpu.VMEM((_CHUNK,), jnp.float32),
        pltpu.VMEM((_CHUNK,), jnp.float32),
        pltpu.VMEM((_CHUNK,), jnp.float32),
        pltpu.VMEM((3 * _CHUNK,), jnp.int32),
        pltpu.VMEM((3 * _CHUNK,), jnp.float32),
        pltpu.VMEM((4 * _CHUNK,), jnp.float32),
    ]
    sc_fn = pl.kernel(
        body,
        out_type=jax.ShapeDtypeStruct((4 * n,), jnp.float32),
        mesh=mesh,
        compiler_params=pltpu.CompilerParams(
            needs_layout_passes=False, use_tc_tiling_on_sc=False),
        scratch_types=(
            buf_set + buf_set
            + [pltpu.VMEM((9 * _L,), jnp.float32)]
            + [pltpu.SemaphoreType.DMA] * 6
        ),
    )
    out_phys = sc_fn(u.astype(jnp.int32), v.astype(jnp.int32),
                     cP[0], cP[1], cP[2], j_flat, consts)
    return (out_phys.reshape(n // 128, 4, 128)
            .transpose(1, 0, 2)
            .reshape(4, n)[:3])
